# Initial kernel scaffold; baseline (speedup 1.0000x reference)
import jax
import jax.numpy as jnp
from jax.experimental import pallas as pl


def kernel(z, codebook, ema_cluster_size):
    raise NotImplementedError("write your pallas kernel here")


# speedup vs baseline: 1.0787x; 1.0787x over previous
import functools

import jax
import jax.numpy as jnp
from jax.experimental import pallas as pl
from jax.experimental.pallas import tpu as pltpu
from jax.experimental.pallas import tpu_sc as plsc

K = 8192
D = 32
BETA = 0.25
EPS = 1e-5
EOP_TOKEN_ID = 3
PADDING_TOKEN_ID = 2

_RB = 256
_G = 8192 // _RB

_NW = 32
_BPW = 8192 // _NW


_CH = 2048


def _argmin_body(x_ref, ct_ref, x2_ref, c2_ref, idx_ref):
    dot2 = jnp.dot(x_ref[...], ct_ref[...], preferred_element_type=jnp.float32)
    dist = (x2_ref[...] + dot2) + c2_ref[...]
    acc_v = acc_i = None
    for c in range(K // _CH):
        dc = dist[:, c * _CH:(c + 1) * _CH]
        mv = jnp.min(dc, axis=1)
        mi = jnp.argmin(dc, axis=1).astype(jnp.int32) + c * _CH
        if c == 0:
            acc_v = mv.astype(jnp.bfloat16).astype(jnp.float32)
            acc_i = mi
        else:
            take = mv < acc_v
            acc_v = jnp.where(take, mv, acc_v).astype(
                jnp.bfloat16).astype(jnp.float32)
            acc_i = jnp.where(take, mi, acc_i)
    idx_ref[...] = acc_i.reshape(1, 1, _RB)


def _argmin_call(flat, ct, x2, c2):
    return pl.pallas_call(
        _argmin_body,
        grid=(_G,),
        in_specs=[
            pl.BlockSpec((_RB, D), lambda i: (i, 0)),
            pl.BlockSpec((D, K), lambda i: (0, 0)),
            pl.BlockSpec((_RB, 1), lambda i: (i, 0)),
            pl.BlockSpec((1, K), lambda i: (0, 0)),
        ],
        out_specs=pl.BlockSpec((1, 1, _RB), lambda i: (i, 0, 0)),
        out_shape=jax.ShapeDtypeStruct((_G, 1, _RB), jnp.int32),
        compiler_params=pltpu.CompilerParams(
            dimension_semantics=("arbitrary",)),
    )(flat, ct, x2, c2)


def _sc_gather(table_pad, idx_flat):
    mesh = plsc.VectorSubcoreMesh(core_axis_name="c", subcore_axis_name="s")

    @functools.partial(
        pl.kernel,
        mesh=mesh,
        out_type=jax.ShapeDtypeStruct((K, 128), jnp.float32),
        scratch_types=[
            pltpu.VMEM((_BPW,), jnp.int32),
            pltpu.VMEM((_BPW, 128), jnp.float32),
            pltpu.SemaphoreType.DMA,
        ],
    )
    def k(table_hbm, idx_hbm, out_hbm, idx_v, rows_v, sem):
        wid = jax.lax.axis_index("s") * 2 + jax.lax.axis_index("c")
        base = wid * _BPW
        pltpu.sync_copy(idx_hbm.at[pl.ds(base, _BPW)], idx_v)
        pltpu.async_copy(table_hbm.at[idx_v], rows_v, sem).wait()
        pltpu.sync_copy(rows_v, out_hbm.at[pl.ds(base, _BPW)])

    return k(table_pad, idx_flat)


def _epilogue_body(z_ref, zq_ref, ema_ref, zqste_ref, loss_ref, perp_ref):
    z = z_ref[...]
    d = zq_ref[:, :D] - z
    zqste_ref[...] = z + d
    m = jnp.mean(d * d)
    loss_ref[...] = (m + BETA * m).reshape(1, 1)

    ema = ema_ref[...]
    r = jax.lax.broadcasted_iota(jnp.int32, ema.shape, 0)
    c = jax.lax.broadcasted_iota(jnp.int32, ema.shape, 1)
    fi = r * ema.shape[1] + c
    counts = jnp.where((fi == EOP_TOKEN_ID) | (fi == PADDING_TOKEN_ID),
                       0.0, ema)
    counts = jnp.maximum(counts, EPS)
    probs = counts / (jnp.sum(counts) + EPS)
    entropy = -jnp.sum(probs * jnp.log(probs))
    perp_ref[...] = jnp.exp(entropy).reshape(1, 1)


def _epilogue_call(flat, zq, ema2d):
    return pl.pallas_call(
        _epilogue_body,
        out_shape=(
            jax.ShapeDtypeStruct((K, D), jnp.float32),
            jax.ShapeDtypeStruct((1, 1), jnp.float32),
            jax.ShapeDtypeStruct((1, 1), jnp.float32),
        ),
    )(flat, zq, ema2d)


def kernel(z, codebook, ema_cluster_size):
    B, Q, D_in = z.shape
    flat = z.reshape(-1, D_in)
    x2 = jnp.sum(flat ** 2, axis=1, keepdims=True)
    c2 = jnp.sum(codebook ** 2, axis=1).reshape(1, K)
    xr = flat.astype(jnp.bfloat16).astype(jnp.float32)
    ct = (-2.0 * codebook).astype(jnp.bfloat16).astype(jnp.float32).T
    idx3 = _argmin_call(xr, ct, x2, c2)
    idx_flat = idx3.reshape(-1)
    table_pad = jnp.pad(codebook, ((0, 0), (0, 128 - D)))
    zq = _sc_gather(table_pad, idx_flat)
    zqste, loss, perp = _epilogue_call(flat, zq,
                                       ema_cluster_size.reshape(8, K // 8))
    return (zqste.reshape(B, Q, D_in), loss[0, 0],
            idx_flat.reshape(B, Q), perp[0, 0])
